# baseline (speedup 1.0000x reference)
import functools

import jax
import jax.numpy as jnp
from jax import lax
from jax.experimental import pallas as pl
from jax.experimental.pallas import tpu as pltpu
from jax.experimental.pallas import tpu_sc as plsc

EMB = 16
N_EDGES = 1600000
CHUNK = 1280
NUM_CHUNKS = N_EDGES // CHUNK
NW = 32
L = 16
NBUF = 2


def _sc_lookup(idx0, idx1, Wc):
    mesh = plsc.VectorSubcoreMesh(core_axis_name="c", subcore_axis_name="s")

    @functools.partial(
        pl.kernel,
        mesh=mesh,
        compiler_params=pltpu.CompilerParams(use_tc_tiling_on_sc=False),
        out_type=jax.ShapeDtypeStruct((N_EDGES, 2 * EMB), jnp.float32),
        scratch_types=[
            [pltpu.VMEM((CHUNK,), jnp.int32) for _ in range(NBUF)],
            [pltpu.VMEM((CHUNK,), jnp.int32) for _ in range(NBUF)],
            [pltpu.VMEM((CHUNK,), jnp.int32) for _ in range(NBUF)],
            [pltpu.VMEM((CHUNK, 2 * EMB), jnp.float32) for _ in range(NBUF)],
            [pltpu.SemaphoreType.DMA for _ in range(NBUF)],
            [pltpu.SemaphoreType.DMA for _ in range(NBUF)],
            [pltpu.SemaphoreType.DMA for _ in range(NBUF)],
        ],
    )
    def k(idx0_hbm, idx1_hbm, wc_hbm, out_hbm,
          i0_v, i1_v, ci_v, out_v, isem, gsem, wsem):
        wid = lax.axis_index("s") * 2 + lax.axis_index("c")
        steps = (NUM_CHUNKS + NW - 1) // NW
        tw = lax.div(NUM_CHUNKS - wid + NW - 1, NW)

        def start_idx(t, b):
            base = (wid + t * NW) * CHUNK
            pltpu.async_copy(idx0_hbm.at[pl.ds(base, CHUNK)], i0_v[b], isem[b])
            pltpu.async_copy(idx1_hbm.at[pl.ds(base, CHUNK)], i1_v[b], isem[b])

        def wait_idx(b):
            pltpu.make_async_copy(
                idx0_hbm.at[pl.ds(0, CHUNK)], i0_v[b], isem[b]).wait()
            pltpu.make_async_copy(
                idx1_hbm.at[pl.ds(0, CHUNK)], i1_v[b], isem[b]).wait()

        def wait_write(b):
            pltpu.make_async_copy(
                out_v[b], out_hbm.at[pl.ds(0, CHUNK), :], wsem[b]).wait()

        def run_chunk(t, b):
            wait_idx(b)
            for o in range(0, CHUNK, L):
                ci_v[b][pl.ds(o, L)] = (
                    i0_v[b][pl.ds(o, L)] * 4 + i1_v[b][pl.ds(o, L)])
            cps = []
            for j in range(0, CHUNK, 128):
                cps.append(pltpu.async_copy(
                    wc_hbm.at[ci_v[b].at[pl.ds(j, 128)]],
                    out_v[b].at[pl.ds(j, 128), :], gsem[b]))
            for cp in cps:
                cp.wait()
            base = (wid + t * NW) * CHUNK
            pltpu.async_copy(out_v[b], out_hbm.at[pl.ds(base, CHUNK), :], wsem[b])

        start_idx(0, 0)

        def body(t, carry):
            for bb in range(NBUF):
                @pl.when(lax.rem(t, NBUF) == bb)
                def _(bb=bb):
                    @pl.when(t + 1 < tw)
                    def _():
                        start_idx(t + 1, (bb + 1) % NBUF)

                    @pl.when(t < tw)
                    def _():
                        @pl.when(t >= NBUF)
                        def _():
                            wait_write(bb)
                        run_chunk(t, bb)
            return carry

        lax.fori_loop(0, steps, body, 0)

        for kk in range(NBUF):
            tp = tw - 1 - kk
            for bb in range(NBUF):
                @pl.when(jnp.logical_and(tp >= 0, lax.rem(tp, NBUF) == bb))
                def _(bb=bb):
                    wait_write(bb)

    return k(idx0, idx1, Wc)


def kernel(edge_attr, W0, W1):
    idx0 = edge_attr[:, 0]
    idx1 = edge_attr[:, 1]
    Wc = jnp.concatenate(
        [jnp.repeat(W0, 4, axis=0), jnp.tile(W1, (4, 1))], axis=1)
    return _sc_lookup(idx0, idx1, Wc)


# speedup vs baseline: 3.6583x; 3.6583x over previous
import functools

import jax
import jax.numpy as jnp
from jax import lax
from jax.experimental import pallas as pl
from jax.experimental.pallas import tpu as pltpu
from jax.experimental.pallas import tpu_sc as plsc

EMB = 16
N_EDGES = 1600000
CHUNK = 1280
NUM_CHUNKS = N_EDGES // CHUNK
NW = 32
L = 16
NBUF = 2


def _sc_lookup(idx0, idx1, Wc):
    mesh = plsc.VectorSubcoreMesh(core_axis_name="c", subcore_axis_name="s")

    @functools.partial(
        pl.kernel,
        mesh=mesh,
        compiler_params=pltpu.CompilerParams(use_tc_tiling_on_sc=False),
        out_type=jax.ShapeDtypeStruct((N_EDGES, 2 * EMB), jnp.float32),
        scratch_types=[
            [pltpu.VMEM((CHUNK,), jnp.int32) for _ in range(NBUF)],
            [pltpu.VMEM((CHUNK,), jnp.int32) for _ in range(NBUF)],
            [pltpu.VMEM((CHUNK,), jnp.int32) for _ in range(NBUF)],
            [pltpu.VMEM((CHUNK, 2 * EMB), jnp.float32) for _ in range(NBUF)],
            [pltpu.SemaphoreType.DMA for _ in range(NBUF)],
            [pltpu.SemaphoreType.DMA for _ in range(NBUF)],
            [pltpu.SemaphoreType.DMA for _ in range(NBUF)],
        ],
    )
    def k(idx0_hbm, idx1_hbm, wc_hbm, out_hbm,
          i0_v, i1_v, ci_v, out_v, isem, gsem, wsem):
        wid = lax.axis_index("s") * 2 + lax.axis_index("c")
        steps = (NUM_CHUNKS + NW - 1) // NW
        tw = lax.div(NUM_CHUNKS - wid + NW - 1, NW)

        def start_idx(t, b):
            base = (wid + t * NW) * CHUNK
            pltpu.async_copy(idx0_hbm.at[pl.ds(base, CHUNK)], i0_v[b], isem[b])
            pltpu.async_copy(idx1_hbm.at[pl.ds(base, CHUNK)], i1_v[b], isem[b])

        def wait_idx(b):
            pltpu.make_async_copy(
                idx0_hbm.at[pl.ds(0, CHUNK)], i0_v[b], isem[b]).wait()
            pltpu.make_async_copy(
                idx1_hbm.at[pl.ds(0, CHUNK)], i1_v[b], isem[b]).wait()

        def wait_write(b):
            pltpu.make_async_copy(
                out_v[b], out_hbm.at[pl.ds(0, CHUNK), :], wsem[b]).wait()

        def run_chunk(t, b):
            wait_idx(b)
            rep = wid * 16
            for o in range(0, CHUNK, L):
                ci_v[b][pl.ds(o, L)] = (
                    i0_v[b][pl.ds(o, L)] * 4 + i1_v[b][pl.ds(o, L)] + rep)
            cps = []
            for j in range(0, CHUNK, 128):
                cps.append(pltpu.async_copy(
                    wc_hbm.at[ci_v[b].at[pl.ds(j, 128)]],
                    out_v[b].at[pl.ds(j, 128), :], gsem[b]))
            for cp in cps:
                cp.wait()
            base = (wid + t * NW) * CHUNK
            pltpu.async_copy(out_v[b], out_hbm.at[pl.ds(base, CHUNK), :], wsem[b])

        start_idx(0, 0)

        def body(t, carry):
            for bb in range(NBUF):
                @pl.when(lax.rem(t, NBUF) == bb)
                def _(bb=bb):
                    @pl.when(t + 1 < tw)
                    def _():
                        start_idx(t + 1, (bb + 1) % NBUF)

                    @pl.when(t < tw)
                    def _():
                        @pl.when(t >= NBUF)
                        def _():
                            wait_write(bb)
                        run_chunk(t, bb)
            return carry

        lax.fori_loop(0, steps, body, 0)

        for kk in range(NBUF):
            tp = tw - 1 - kk
            for bb in range(NBUF):
                @pl.when(jnp.logical_and(tp >= 0, lax.rem(tp, NBUF) == bb))
                def _(bb=bb):
                    wait_write(bb)

    return k(idx0, idx1, Wc)


def kernel(edge_attr, W0, W1):
    idx0 = edge_attr[:, 0]
    idx1 = edge_attr[:, 1]
    Wc = jnp.concatenate(
        [jnp.repeat(W0, 4, axis=0), jnp.tile(W1, (4, 1))], axis=1)
    Wc_rep = jnp.tile(Wc, (NW, 1))
    return _sc_lookup(idx0, idx1, Wc_rep)
